# baseline (speedup 1.0000x reference)
import functools

import jax
import jax.numpy as jnp
from jax import lax
from jax.experimental import pallas as pl
from jax.experimental.pallas import tpu as pltpu
from jax.experimental.pallas import tpu_sc as plsc

BATCH = 4096
SEQ = 200
EMBED_DIM = 64
NUM_WORKERS = 32
SEQ_PER_WORKER = BATCH // NUM_WORKERS
HALF = SEQ // 2


def _emb_body(idx_hbm, emb_hbm, pos_hbm, out_hbm, idx_v, rows_v, pos_v, sem):
    num_cores = 2
    wid = lax.axis_index("s") * num_cores + lax.axis_index("c")

    pltpu.sync_copy(pos_hbm, pos_v)

    def seq_body(s, carry):
        seq = wid * SEQ_PER_WORKER + s
        base = seq * SEQ
        pltpu.sync_copy(idx_hbm.at[pl.ds(2 * seq, 2)], idx_v)
        c0 = pltpu.async_copy(
            emb_hbm.at[idx_v.at[0]], rows_v.at[pl.ds(0, HALF)], sem
        )
        c1 = pltpu.async_copy(
            emb_hbm.at[idx_v.at[1]], rows_v.at[pl.ds(HALF, HALF)], sem
        )
        c0.wait()
        c1.wait()

        def add_row(r, carry2):
            for c in range(EMBED_DIM // 16):
                sl = pl.ds(c * 16, 16)
                rows_v[r, sl] = rows_v[r, sl] + pos_v[r, sl]
            return carry2

        lax.fori_loop(0, SEQ, add_row, 0, unroll=2)

        pltpu.sync_copy(rows_v, out_hbm.at[pl.ds(base, SEQ)])
        return carry

    lax.fori_loop(0, SEQ_PER_WORKER, seq_body, 0)


@functools.partial(jax.jit, static_argnames=())
def kernel(x, emb_table, pos_table):
    idx = x.reshape(BATCH * SEQ // HALF, HALF).astype(jnp.int32)
    mesh = plsc.VectorSubcoreMesh(core_axis_name="c", subcore_axis_name="s")
    run = pl.kernel(
        _emb_body,
        out_type=jax.ShapeDtypeStruct((BATCH * SEQ, EMBED_DIM), jnp.float32),
        mesh=mesh,
        scratch_types=[
            pltpu.VMEM((2, HALF), jnp.int32),
            pltpu.VMEM((SEQ, EMBED_DIM), jnp.float32),
            pltpu.VMEM((SEQ, EMBED_DIM), jnp.float32),
            pltpu.SemaphoreType.DMA,
        ],
        compiler_params=pltpu.CompilerParams(use_tc_tiling_on_sc=False),
    )
    out = run(idx, emb_table, pos_table)
    return out.reshape(BATCH, SEQ, EMBED_DIM)
